# Initial kernel scaffold; baseline (speedup 1.0000x reference)
import jax
import jax.numpy as jnp
from jax.experimental import pallas as pl


def kernel(x, edge_index, edge_attr, batch, We1, be1, W11, b11, W12, b12, We2, be2, W21, b21, W22, b22):
    raise NotImplementedError("write your pallas kernel here")


# speedup vs baseline: 1.4025x; 1.4025x over previous
import functools

import jax
import jax.numpy as jnp
from jax import lax
from jax.experimental import pallas as pl
from jax.experimental.pallas import tpu as pltpu
from jax.experimental.pallas import tpu_sc as plsc

N = 10000
E = 320000
DIN = 128
DH = 256
G = 64

NC = 2
NS = 16
L = 16

C = 80
EPS = E // NS
NCHUNK = EPS // C
NPS = N // NS
ZROWS = 125
WOUT = 624
WLAST = N - (NS - 1) * WOUT


def _make_agg(D2, split_features):
  mesh = plsc.VectorSubcoreMesh(
      core_axis_name="c", subcore_axis_name="s", num_cores=NC, num_subcores=NS)
  eps = EPS if split_features else EPS // NC
  nchunk = eps // C

  def body(table_hbm, src_hbm, dst_hbm, attr_hbm, we_hbm, be_hbm, out_hbm,
           src_v, gidx_v, dst_v, a_v, rows_v, zbuf_v, we_v, be_v, acc_sh, sem):
    c = lax.axis_index("c")
    s = lax.axis_index("s")

    widx = c if split_features else 0
    pltpu.sync_copy(we_hbm.at[widx], we_v)
    pltpu.sync_copy(be_hbm.at[widx], be_v)

    def zrow(i, _):
      for k in range(D2 // L):
        zbuf_v[i, pl.ds(k * L, L)] = jnp.zeros((L,), jnp.float32)
      return 0
    lax.fori_loop(0, ZROWS, zrow, 0)
    for j in range(NPS // ZROWS):
      pltpu.sync_copy(zbuf_v, acc_sh.at[pl.ds(s * NPS + j * ZROWS, ZROWS)])
    plsc.subcore_barrier()

    base0 = (s * NC + c) * eps if not split_features else s * eps

    def chunk(i, _):
      base = base0 + i * C
      pltpu.sync_copy(src_hbm.at[pl.ds(base, C)], src_v)
      pltpu.sync_copy(dst_hbm.at[pl.ds(base, C)], dst_v)
      pltpu.sync_copy(attr_hbm.at[pl.ds(base, C)], a_v)
      if split_features:
        for k in range(C // L):
          sl = pl.ds(k * L, L)
          gidx_v[sl] = src_v[sl] * 2 + c
        idx_ref = gidx_v
      else:
        idx_ref = src_v
      pltpu.async_copy(table_hbm.at[idx_ref], rows_v, sem).wait()

      def grp(g, _):
        a16 = a_v[pl.ds(g * L, L)]
        for j in range(L):
          e = g * L + j
          a_b = a16[j]
          for k in range(D2 // L):
            sl = pl.ds(k * L, L)
            rows_v[e, sl] = jnp.maximum(
                rows_v[e, sl] + a_b * we_v[sl] + be_v[sl], 0.0)
        return 0
      lax.fori_loop(0, C // L, grp, 0)

      pltpu.sync_copy(rows_v, acc_sh.at[dst_v], add=True)
      return 0
    lax.fori_loop(0, nchunk, chunk, 0)

    plsc.subcore_barrier()

    @pl.when(s < NS - 1)
    def _():
      pltpu.sync_copy(acc_sh.at[pl.ds(s * WOUT, WOUT)],
                      out_hbm.at[c, pl.ds(s * WOUT, WOUT)])

    @pl.when(s == NS - 1)
    def _():
      pltpu.sync_copy(acc_sh.at[pl.ds((NS - 1) * WOUT, WLAST)],
                      out_hbm.at[c, pl.ds((NS - 1) * WOUT, WLAST)])

  return pl.kernel(
      body,
      out_type=jax.ShapeDtypeStruct((NC, N, D2), jnp.float32),
      mesh=mesh,
      scratch_types=[
          pltpu.VMEM((C,), jnp.int32),
          pltpu.VMEM((C,), jnp.int32),
          pltpu.VMEM((C,), jnp.int32),
          pltpu.VMEM((C,), jnp.float32),
          pltpu.VMEM((C, D2), jnp.float32),
          pltpu.VMEM((ZROWS, D2), jnp.float32),
          pltpu.VMEM((D2,), jnp.float32),
          pltpu.VMEM((D2,), jnp.float32),
          pltpu.VMEM_SHARED((N, D2), jnp.float32),
          pltpu.SemaphoreType.DMA,
      ],
  )


_agg1 = _make_agg(DIN, split_features=False)
_agg2 = _make_agg(DH // 2, split_features=True)

NB = 400
NBLK = N // NB


def _mlp1_body(x_ref, a0_ref, a1_ref, w1_ref, b1_ref, w2_ref, b2_ref, o_ref):
  z = x_ref[...] + a0_ref[...] + a1_ref[...]
  t = jnp.maximum(jnp.dot(z, w1_ref[...], preferred_element_type=jnp.float32)
                  + b1_ref[...], 0.0)
  h = jnp.dot(t, w2_ref[...], preferred_element_type=jnp.float32) + b2_ref[...]
  o_ref[...] = jnp.maximum(h, 0.0)


def _mlp1(x, a0, a1, w1, b1, w2, b2):
  return pl.pallas_call(
      _mlp1_body,
      grid=(NBLK,),
      in_specs=[
          pl.BlockSpec((NB, DIN), lambda i: (i, 0)),
          pl.BlockSpec((NB, DIN), lambda i: (i, 0)),
          pl.BlockSpec((NB, DIN), lambda i: (i, 0)),
          pl.BlockSpec((DIN, DH), lambda i: (0, 0)),
          pl.BlockSpec((1, DH), lambda i: (0, 0)),
          pl.BlockSpec((DH, DH), lambda i: (0, 0)),
          pl.BlockSpec((1, DH), lambda i: (0, 0)),
      ],
      out_specs=pl.BlockSpec((NB, DH), lambda i: (i, 0)),
      out_shape=jax.ShapeDtypeStruct((N, DH), jnp.float32),
  )(x, a0, a1, w1, b1, w2, b2)


def _mlp2_pool_body(h_ref, a0_ref, a1_ref, b_ref, w1_ref, b1_ref, w2_ref,
                    b2_ref, o_ref, acc_ref, cnt_ref):
  i = pl.program_id(0)

  @pl.when(i == 0)
  def _():
    acc_ref[...] = jnp.zeros_like(acc_ref)
    cnt_ref[...] = jnp.zeros_like(cnt_ref)

  z = h_ref[...] + jnp.concatenate([a0_ref[...], a1_ref[...]], axis=1)
  t = jnp.maximum(jnp.dot(z, w1_ref[...], preferred_element_type=jnp.float32)
                  + b1_ref[...], 0.0)
  h = jnp.maximum(jnp.dot(t, w2_ref[...], preferred_element_type=jnp.float32)
                  + b2_ref[...], 0.0)
  seg = b_ref[0]
  gids = lax.broadcasted_iota(jnp.int32, (G, NB), 0)
  onehot = (gids == seg).astype(jnp.float32)
  acc_ref[...] += jnp.dot(onehot, h, preferred_element_type=jnp.float32)
  cnt_ref[...] += jnp.broadcast_to(
      jnp.sum(onehot, axis=1, keepdims=True), cnt_ref.shape)

  @pl.when(i == NBLK - 1)
  def _():
    cfull = cnt_ref[...]
    c2 = jnp.concatenate([cfull, cfull], axis=1)
    o_ref[...] = acc_ref[...] / jnp.maximum(c2, 1.0)


def _mlp2_pool(h, a0, a1, batch3d, w1, b1, w2, b2):
  return pl.pallas_call(
      _mlp2_pool_body,
      grid=(NBLK,),
      in_specs=[
          pl.BlockSpec((NB, DH), lambda i: (i, 0)),
          pl.BlockSpec((NB, DH // 2), lambda i: (i, 0)),
          pl.BlockSpec((NB, DH // 2), lambda i: (i, 0)),
          pl.BlockSpec((1, 1, NB), lambda i: (i, 0, 0)),
          pl.BlockSpec((DH, DH), lambda i: (0, 0)),
          pl.BlockSpec((1, DH), lambda i: (0, 0)),
          pl.BlockSpec((DH, DH), lambda i: (0, 0)),
          pl.BlockSpec((1, DH), lambda i: (0, 0)),
      ],
      out_specs=pl.BlockSpec((G, DH), lambda i: (0, 0)),
      out_shape=jax.ShapeDtypeStruct((G, DH), jnp.float32),
      scratch_shapes=[
          pltpu.VMEM((G, DH), jnp.float32),
          pltpu.VMEM((G, 128), jnp.float32),
      ],
  )(h, a0, a1, batch3d, w1, b1, w2, b2)


@jax.jit
def kernel(x, edge_index, edge_attr, batch, We1, be1, W11, b11, W12, b12,
           We2, be2, W21, b21, W22, b22):
  src = edge_index[0]
  dst = edge_index[1]
  attr = edge_attr.reshape(E)

  aggr1 = _agg1(x, src, dst, attr,
                We1.reshape(1, DIN), be1.reshape(1, DIN))
  h1 = _mlp1(x, aggr1[0], aggr1[1], W11, b11.reshape(1, DH),
             W12, b12.reshape(1, DH))

  aggr2 = _agg2(h1.reshape(2 * N, DH // 2), src, dst, attr,
                We2.reshape(NC, DH // 2), be2.reshape(NC, DH // 2))
  out = _mlp2_pool(h1, aggr2[0], aggr2[1], batch.reshape(NBLK, 1, NB),
                   W21, b21.reshape(1, DH), W22, b22.reshape(1, DH))
  return out
